# baseline (speedup 1.0000x reference)
import functools

import jax
import jax.numpy as jnp
from jax import lax
from jax.experimental import pallas as pl
from jax.experimental.pallas import tpu as pltpu
from jax.experimental.pallas import tpu_sc as plsc

_MARGIN = 12.0
_DIM = 128
_DPAD = 1024
_RPAD = 256
_B = 4096
_NNEG = 32

_TBL_ROWS = 12288
_OFF_RT = 8192 * 128
_OFF_A = 10240 * 128

_NC, _NS = 2, 16
_NW = _NC * _NS
_BPW = _B // _NW
_GPW = _BPW * _NNEG
_SH_LEN = _TBL_ROWS * 128
_SH_SLICE = _SH_LEN // _NS

_RSQRT_MAGIC = 0x5F3759DF


def _prep_body(drug_ref, rel_ref, tgt_ref, tbl_ref):
    d = drug_ref[...]
    r = rel_ref[...]
    t = tgt_ref[...]
    dn = jnp.sum(d * d, axis=1, keepdims=True)
    rn = jnp.sum(r * r, axis=1, keepdims=True)
    tn = jnp.sum(t * t, axis=1, keepdims=True)
    ones_d = jnp.ones((_DPAD, 1), jnp.float32)
    ones_r = jnp.ones((_RPAD, 1), jnp.float32)
    dot = functools.partial(
        lax.dot_general, dimension_numbers=(((1,), (1,)), ((), ())),
        preferred_element_type=jnp.float32)
    d1 = jnp.concatenate([d, ones_d], axis=1)
    t1 = jnp.concatenate([-2.0 * t, tn], axis=1)
    dt = dot(d1, t1)
    rt = dot(-2.0 * r, t)
    d2 = jnp.concatenate([d, dn, ones_d], axis=1)
    r2 = jnp.concatenate([2.0 * r, ones_r, rn], axis=1)
    at = dot(r2, d2)
    for k in range(8):
        lo, hi = k * 128, (k + 1) * 128
        tbl_ref[pl.ds(k * 1024, 1024), :] = dt[:, lo:hi]
        tbl_ref[pl.ds(8192 + k * 256, 256), :] = rt[:, lo:hi]
        tbl_ref[pl.ds(10240 + k * 256, 256), :] = at[:, lo:hi]


def _epi_body(g1_ref, g2_ref, g3_ref, out_ref):
    s = g1_ref[...] + g2_ref[...]
    st = lax.transpose(s, (1, 0))
    ga = lax.transpose(jnp.squeeze(g3_ref[...], 0), (1, 0))
    out_ref[...] = _MARGIN - jnp.sqrt(jnp.maximum(st + ga, 0.0))


@functools.partial(
    pl.kernel,
    mesh=plsc.VectorSubcoreMesh(core_axis_name="c", subcore_axis_name="s"),
    out_type=[
        jax.ShapeDtypeStruct((_B * _NNEG,), jnp.float32),
        jax.ShapeDtypeStruct((_B * _NNEG,), jnp.float32),
        jax.ShapeDtypeStruct((_B,), jnp.float32),
    ],
    scratch_types=[
        pltpu.VMEM_SHARED((_SH_LEN,), jnp.float32),
        pltpu.VMEM((_BPW,), jnp.int32),
        pltpu.VMEM((_BPW,), jnp.int32),
        pltpu.VMEM((_GPW,), jnp.int32),
        pltpu.VMEM((_GPW,), jnp.int32),
        pltpu.VMEM((_GPW,), jnp.int32),
        pltpu.VMEM((_BPW,), jnp.int32),
        pltpu.VMEM((_GPW,), jnp.float32),
        pltpu.VMEM((_GPW,), jnp.float32),
        pltpu.VMEM((_BPW,), jnp.float32),
        pltpu.SemaphoreType.DMA,
        pltpu.SemaphoreType.DMA,
        pltpu.SemaphoreType.DMA,
        pltpu.SemaphoreType.DMA,
    ],
)
def _sc_main(tbl_hbm, u_hbm, r_hbm, vt_hbm, o1_hbm, o2_hbm, o3_hbm,
             tbl_sh, u_v, r_v, vt_v, i1_v, i2_v, i3_v,
             g1_v, g2_v, g3_v, sem, semb, sem2, sem3):
    sid = lax.axis_index("s")
    wid = sid * _NC + lax.axis_index("c")
    rb = wid * _BPW
    gb = wid * _GPW
    s1 = _OFF_RT // _NS
    s2 = (_SH_LEN - _OFF_RT) // _NS
    stg1 = pltpu.make_async_copy(tbl_hbm.at[pl.ds(sid * s1, s1)],
                                 tbl_sh.at[pl.ds(sid * s1, s1)], sem2)
    stg2 = pltpu.make_async_copy(
        tbl_hbm.at[pl.ds(_OFF_RT + sid * s2, s2)],
        tbl_sh.at[pl.ds(_OFF_RT + sid * s2, s2)], sem3)
    stg1.start()
    stg2.start()
    pltpu.sync_copy(u_hbm.at[pl.ds(rb, _BPW)], u_v)
    pltpu.sync_copy(r_hbm.at[pl.ds(rb, _BPW)], r_v)
    pltpu.sync_copy(vt_hbm.at[pl.ds(gb, _GPW)], vt_v)

    def idx_body(i, carry):
        mb = lax.shift_left(jnp.bitwise_and(i, 7), 4)
        u16 = u_v[pl.ds(mb, 16)]
        r16 = r_v[pl.ds(mb, 16)]
        p = i * 16
        v16 = vt_v[pl.ds(p, 16)]
        vhi = lax.shift_right_logical(v16, 7)
        vlo = jnp.bitwise_and(v16, 127)
        i1_v[pl.ds(p, 16)] = (
            lax.shift_left(vhi, 17) + lax.shift_left(u16, 7) + vlo)
        i2_v[pl.ds(p, 16)] = (
            _OFF_RT + lax.shift_left(vhi, 15)
            + lax.shift_left(r16, 7) + vlo)
        return carry

    lax.fori_loop(0, _GPW // 16, idx_body, 0)

    def ia_body(m, carry):
        mb = m * 16
        u16 = u_v[pl.ds(mb, 16)]
        r16 = r_v[pl.ds(mb, 16)]
        i3_v[pl.ds(mb, 16)] = (
            _OFF_A + lax.shift_left(lax.shift_right_logical(u16, 7), 15)
            + lax.shift_left(r16, 7) + jnp.bitwise_and(u16, 127))
        return carry

    lax.fori_loop(0, _BPW // 16, ia_body, 0)

    half = _GPW // 2
    stg1.wait()
    plsc.subcore_barrier()
    c1a = pltpu.make_async_copy(tbl_sh.at[i1_v.at[pl.ds(0, half)]],
                                g1_v.at[pl.ds(0, half)], sem)
    c1b = pltpu.make_async_copy(tbl_sh.at[i1_v.at[pl.ds(half, half)]],
                                g1_v.at[pl.ds(half, half)], semb)
    c1a.start()
    c1b.start()
    stg2.wait()
    plsc.subcore_barrier()
    c2a = pltpu.make_async_copy(tbl_sh.at[i2_v.at[pl.ds(0, half)]],
                                g2_v.at[pl.ds(0, half)], sem)
    c2b = pltpu.make_async_copy(tbl_sh.at[i2_v.at[pl.ds(half, half)]],
                                g2_v.at[pl.ds(half, half)], semb)
    c3 = pltpu.make_async_copy(tbl_sh.at[i3_v], g3_v, sem)
    c2a.start()
    c2b.start()
    c3.start()

    c1a.wait()
    c2a.wait()
    c3.wait()
    pltpu.sync_copy(g1_v.at[pl.ds(0, half)], o1_hbm.at[pl.ds(gb, half)])
    pltpu.sync_copy(g2_v.at[pl.ds(0, half)], o2_hbm.at[pl.ds(gb, half)])
    pltpu.sync_copy(g3_v, o3_hbm.at[pl.ds(rb, _BPW)])
    c1b.wait()
    c2b.wait()
    pltpu.sync_copy(g1_v.at[pl.ds(half, half)],
                    o1_hbm.at[pl.ds(gb + half, half)])
    pltpu.sync_copy(g2_v.at[pl.ds(half, half)],
                    o2_hbm.at[pl.ds(gb + half, half)])


def kernel(u_idx, r_idx, v_idx, drug_emb, rel_emb, target_emb):
    u = u_idx
    r = r_idx
    v = v_idx

    tbl2d = pl.pallas_call(
        _prep_body,
        grid=(1,),
        in_specs=[
            pl.BlockSpec((_DPAD, _DIM), lambda i: (0, 0)),
            pl.BlockSpec((_RPAD, _DIM), lambda i: (0, 0)),
            pl.BlockSpec((_DPAD, _DIM), lambda i: (0, 0)),
        ],
        out_specs=pl.BlockSpec((_TBL_ROWS, 128), lambda i: (0, 0)),
        out_shape=jax.ShapeDtypeStruct((_TBL_ROWS, 128), jnp.float32),
    )(drug_emb, rel_emb, target_emb)

    vt = v.reshape(_NW, _BPW, _NNEG).transpose(0, 2, 1).reshape(-1)
    g1, g2, g3 = _sc_main(tbl2d.reshape(-1), u, r, vt)

    return pl.pallas_call(
        _epi_body,
        grid=(_NW,),
        in_specs=[
            pl.BlockSpec((_NNEG, _BPW), lambda w: (w, 0)),
            pl.BlockSpec((_NNEG, _BPW), lambda w: (w, 0)),
            pl.BlockSpec((1, 1, _BPW), lambda w: (w, 0, 0)),
        ],
        out_specs=pl.BlockSpec((_BPW, _NNEG), lambda w: (w, 0)),
        out_shape=jax.ShapeDtypeStruct((_B, _NNEG), jnp.float32),
    )(g1.reshape(_NW * _NNEG, _BPW), g2.reshape(_NW * _NNEG, _BPW),
      g3.reshape(_NW, 1, _BPW))


# speedup vs baseline: 1.4129x; 1.4129x over previous
import functools

import jax
import jax.numpy as jnp
from jax import lax
from jax.experimental import pallas as pl
from jax.experimental.pallas import tpu as pltpu
from jax.experimental.pallas import tpu_sc as plsc

_MARGIN = 12.0
_DIM = 128
_DPAD = 1024
_RPAD = 256
_B = 4096
_NNEG = 32

_TBL_ROWS = 12288
_OFF_RT = 8192 * 128
_OFF_A = 10240 * 128

_NC, _NS = 2, 16
_NW = _NC * _NS
_BPW = _B // _NW
_GPW = _BPW * _NNEG
_SH_LEN = _TBL_ROWS * 128
_SH_SLICE = _SH_LEN // _NS

_RSQRT_MAGIC = 0x5F3759DF


def _prep_body(drug_ref, rel_ref, tgt_ref, tbl_ref):
    d = drug_ref[...]
    r = rel_ref[...]
    t = tgt_ref[...]
    dn = jnp.sum(d * d, axis=1, keepdims=True)
    rn = jnp.sum(r * r, axis=1, keepdims=True)
    tn = jnp.sum(t * t, axis=1, keepdims=True)
    ones_d = jnp.ones((_DPAD, 1), jnp.float32)
    ones_r = jnp.ones((_RPAD, 1), jnp.float32)
    dot = functools.partial(
        lax.dot_general, dimension_numbers=(((1,), (1,)), ((), ())),
        preferred_element_type=jnp.float32)
    d1 = jnp.concatenate([d, ones_d], axis=1)
    t1 = jnp.concatenate([-2.0 * t, tn], axis=1)
    dt = dot(d1, t1)
    rt = dot(-2.0 * r, t)
    d2 = jnp.concatenate([d, dn, ones_d], axis=1)
    r2 = jnp.concatenate([2.0 * r, ones_r, rn], axis=1)
    at = dot(r2, d2)
    for k in range(8):
        lo, hi = k * 128, (k + 1) * 128
        tbl_ref[pl.ds(k * 1024, 1024), :] = dt[:, lo:hi]
        tbl_ref[pl.ds(8192 + k * 256, 256), :] = rt[:, lo:hi]
        tbl_ref[pl.ds(10240 + k * 256, 256), :] = at[:, lo:hi]


@functools.partial(
    pl.kernel,
    mesh=plsc.VectorSubcoreMesh(core_axis_name="c", subcore_axis_name="s"),
    out_type=jax.ShapeDtypeStruct((_B * _NNEG,), jnp.float32),
    scratch_types=[
        pltpu.VMEM_SHARED((_SH_LEN,), jnp.float32),
        pltpu.VMEM((_BPW,), jnp.int32),
        pltpu.VMEM((_BPW,), jnp.int32),
        pltpu.VMEM((_GPW,), jnp.int32),
        pltpu.VMEM((_GPW,), jnp.int32),
        pltpu.VMEM((_GPW,), jnp.int32),
        pltpu.VMEM((_BPW,), jnp.int32),
        pltpu.VMEM((_GPW,), jnp.float32),
        pltpu.VMEM((_GPW,), jnp.float32),
        pltpu.VMEM((_BPW,), jnp.float32),
        pltpu.SemaphoreType.DMA,
        pltpu.SemaphoreType.DMA,
        pltpu.SemaphoreType.DMA,
        pltpu.SemaphoreType.DMA,
    ],
)
def _sc_main(tbl_hbm, u_hbm, r_hbm, vt_hbm, out_hbm,
             tbl_sh, u_v, r_v, vt_v, i1_v, i2_v, i3_v,
             g1_v, g2_v, g3_v, sem, semb, sem2, sem3):
    sid = lax.axis_index("s")
    wid = sid * _NC + lax.axis_index("c")
    rb = wid * _BPW
    gb = wid * _GPW
    s1 = _OFF_RT // _NS
    s2 = (_SH_LEN - _OFF_RT) // _NS
    stg1 = pltpu.make_async_copy(tbl_hbm.at[pl.ds(sid * s1, s1)],
                                 tbl_sh.at[pl.ds(sid * s1, s1)], sem2)
    stg2 = pltpu.make_async_copy(
        tbl_hbm.at[pl.ds(_OFF_RT + sid * s2, s2)],
        tbl_sh.at[pl.ds(_OFF_RT + sid * s2, s2)], sem3)
    stg1.start()
    stg2.start()
    pltpu.sync_copy(u_hbm.at[pl.ds(rb, _BPW)], u_v)
    pltpu.sync_copy(r_hbm.at[pl.ds(rb, _BPW)], r_v)
    pltpu.sync_copy(vt_hbm.at[pl.ds(gb, _GPW)], vt_v)

    def idx_body(i, carry):
        mb = lax.shift_left(jnp.bitwise_and(i, 7), 4)
        u16 = u_v[pl.ds(mb, 16)]
        r16 = r_v[pl.ds(mb, 16)]
        p = i * 16
        v16 = vt_v[pl.ds(p, 16)]
        vhi = lax.shift_right_logical(v16, 7)
        vlo = jnp.bitwise_and(v16, 127)
        i1_v[pl.ds(p, 16)] = (
            lax.shift_left(vhi, 17) + lax.shift_left(u16, 7) + vlo)
        i2_v[pl.ds(p, 16)] = (
            _OFF_RT + lax.shift_left(vhi, 15)
            + lax.shift_left(r16, 7) + vlo)
        return carry

    lax.fori_loop(0, _GPW // 16, idx_body, 0)

    def ia_body(m, carry):
        mb = m * 16
        u16 = u_v[pl.ds(mb, 16)]
        r16 = r_v[pl.ds(mb, 16)]
        i3_v[pl.ds(mb, 16)] = (
            _OFF_A + lax.shift_left(lax.shift_right_logical(u16, 7), 15)
            + lax.shift_left(r16, 7) + jnp.bitwise_and(u16, 127))
        return carry

    lax.fori_loop(0, _BPW // 16, ia_body, 0)

    half = _GPW // 2
    stg1.wait()
    plsc.subcore_barrier()
    c1a = pltpu.make_async_copy(tbl_sh.at[i1_v.at[pl.ds(0, half)]],
                                g1_v.at[pl.ds(0, half)], sem)
    c1b = pltpu.make_async_copy(tbl_sh.at[i1_v.at[pl.ds(half, half)]],
                                g1_v.at[pl.ds(half, half)], semb)
    c1a.start()
    c1b.start()
    stg2.wait()
    plsc.subcore_barrier()
    c2a = pltpu.make_async_copy(tbl_sh.at[i2_v.at[pl.ds(0, half)]],
                                g2_v.at[pl.ds(0, half)], sem)
    c2b = pltpu.make_async_copy(tbl_sh.at[i2_v.at[pl.ds(half, half)]],
                                g2_v.at[pl.ds(half, half)], semb)
    c3 = pltpu.make_async_copy(tbl_sh.at[i3_v], g3_v, sem)
    c2a.start()
    c2b.start()
    c3.start()

    def fin_body(i, carry):
        mb = lax.shift_left(jnp.bitwise_and(i, 7), 4)
        p = i * 16
        s = (g1_v[pl.ds(p, 16)] + g2_v[pl.ds(p, 16)]
             + g3_v[pl.ds(mb, 16)])
        x = jnp.maximum(s, 1e-12)
        zi = _RSQRT_MAGIC - lax.shift_right_logical(
            lax.bitcast_convert_type(x, jnp.int32), 1)
        z = lax.bitcast_convert_type(zi, jnp.float32)
        z = z * (1.5 - 0.5 * x * z * z)
        g1_v[pl.ds(p, 16)] = _MARGIN - x * z
        return carry

    c1a.wait()
    c2a.wait()
    c3.wait()
    lax.fori_loop(0, half // 16, fin_body, 0)
    c1b.wait()
    c2b.wait()
    lax.fori_loop(half // 16, _GPW // 16, fin_body, 0)
    pltpu.sync_copy(g1_v, out_hbm.at[pl.ds(gb, _GPW)])


def kernel(u_idx, r_idx, v_idx, drug_emb, rel_emb, target_emb):
    u = u_idx
    r = r_idx
    v = v_idx

    tbl2d = pl.pallas_call(
        _prep_body,
        grid=(1,),
        in_specs=[
            pl.BlockSpec((_DPAD, _DIM), lambda i: (0, 0)),
            pl.BlockSpec((_RPAD, _DIM), lambda i: (0, 0)),
            pl.BlockSpec((_DPAD, _DIM), lambda i: (0, 0)),
        ],
        out_specs=pl.BlockSpec((_TBL_ROWS, 128), lambda i: (0, 0)),
        out_shape=jax.ShapeDtypeStruct((_TBL_ROWS, 128), jnp.float32),
    )(drug_emb, rel_emb, target_emb)

    vt = v.reshape(_NW, _BPW, _NNEG).transpose(0, 2, 1).reshape(-1)
    out_jm = _sc_main(tbl2d.reshape(-1), u, r, vt)
    return (out_jm.reshape(_NW, _NNEG, _BPW)
            .transpose(0, 2, 1).reshape(_B, _NNEG))
